# baseline (speedup 1.0000x reference)
import jax
import jax.numpy as jnp
from jax import lax
from jax.experimental import pallas as pl
from jax.experimental.pallas import tpu as pltpu
from jax.experimental.pallas import tpu_sc as plsc

N = 10000
E = 320000
D = 128

NC = 2
NS = 16
LANES = 16
NW = NC * NS
FPW = D // NW
PPW = FPW // 2
CH = 10000
NCHUNK = E // CH
GROUPS = CH // LANES
SHIFT = 14
MASK = (1 << SHIFT) - 1


def _pack_body(x_ref, o_ref):
    xb = x_ref[...].astype(jnp.bfloat16)
    u = lax.bitcast_convert_type(xb, jnp.uint16)
    lo = u[:, : D // 2].astype(jnp.uint32)
    hi = u[:, D // 2:].astype(jnp.uint32)
    word = lax.bitcast_convert_type(lo | (hi << 16), jnp.int32)
    o_ref[...] = word.T


def _pack(x):
    return pl.pallas_call(
        _pack_body,
        out_shape=jax.ShapeDtypeStruct((D // 2, N), jnp.int32),
    )(x)


def _proj_body(h_ref, w_ref, o_ref):
    o_ref[...] = lax.dot_general(
        h_ref[...], w_ref[...], (((0,), (0,)), ((), ())),
        preferred_element_type=jnp.float32)


def _proj(hiT, W):
    return pl.pallas_call(
        _proj_body,
        out_shape=jax.ShapeDtypeStruct((N, D), jnp.float32),
    )(hiT, W)


def _sc_body(xp_hbm, packed_hbm, vals_hbm, out_hbm,
             xcols, acc, pk_b, vals_b, semA, semB):
    w = lax.axis_index("s") * NC + lax.axis_index("c")
    pltpu.sync_copy(xp_hbm.at[pl.ds(w * PPW * N, PPW * N)], xcols)

    zeros = jnp.zeros((LANES,), jnp.float32)

    @plsc.parallel_loop(0, FPW * N // LANES, unroll=8)
    def _zero(i):
        acc[pl.ds(i * LANES, LANES)] = zeros

    poff = [jnp.full((LANES,), p * N, jnp.int32) for p in range(PPW)]
    coff = [jnp.full((LANES,), c * N, jnp.int32) for c in range(FPW)]
    sems = (semA, semB)

    def _start(ck, b, sem):
        off = ck * CH
        pltpu.async_copy(packed_hbm.at[pl.ds(off, CH)],
                         pk_b.at[pl.ds(b * CH, CH)], sem)
        pltpu.async_copy(vals_hbm.at[pl.ds(off, CH)],
                         vals_b.at[pl.ds(b * CH, CH)], sem)

    def _drain(b, sem):
        pltpu.make_async_copy(packed_hbm.at[pl.ds(0, CH)],
                              pk_b.at[pl.ds(b * CH, CH)], sem).wait()
        pltpu.make_async_copy(vals_hbm.at[pl.ds(0, CH)],
                              vals_b.at[pl.ds(b * CH, CH)], sem).wait()

    _start(0, 0, semA)
    _start(1, 1, semB)

    def pair_loop(p, carry):
        for b in range(2):
            ck = p * 2 + b
            sem = sems[b]
            _drain(b, sem)

            @plsc.parallel_loop(0, GROUPS, unroll=8)
            def _group(g):
                base = b * CH + g * LANES
                p16 = pk_b[pl.ds(base, LANES)]
                v16 = vals_b[pl.ds(base, LANES)]
                s16 = p16 & MASK
                d16 = lax.shift_right_logical(p16, SHIFT)
                for cp in range(PPW):
                    gw = plsc.load_gather(xcols, [s16 + poff[cp]])
                    f0 = plsc.bitcast(lax.shift_left(gw, 16), jnp.float32)
                    f1 = plsc.bitcast(gw & jnp.int32(-65536), jnp.float32)
                    plsc.addupdate_scatter(acc, [d16 + coff[cp]],
                                           v16 * f0)
                    plsc.addupdate_scatter(acc, [d16 + coff[cp + PPW]],
                                           v16 * f1)

            nxt = (ck + 2) - NCHUNK * ((ck + 2) // NCHUNK)
            _start(nxt, b, sem)
        return carry

    lax.fori_loop(0, NCHUNK // 2, pair_loop, 0)
    _drain(0, semA)
    _drain(1, semB)
    pltpu.sync_copy(acc.at[pl.ds(0, PPW * N)],
                    out_hbm.at[pl.ds(w * PPW * N, PPW * N)])
    pltpu.sync_copy(acc.at[pl.ds(PPW * N, PPW * N)],
                    out_hbm.at[pl.ds((D // 2 + w * PPW) * N, PPW * N)])


_sc_call = pl.kernel(
    _sc_body,
    out_type=jax.ShapeDtypeStruct((D * N,), jnp.float32),
    mesh=plsc.VectorSubcoreMesh(core_axis_name="c", subcore_axis_name="s",
                                num_cores=NC, num_subcores=NS),
    compiler_params=pltpu.CompilerParams(needs_layout_passes=False),
    scratch_types=[
        pltpu.VMEM((PPW * N,), jnp.int32),
        pltpu.VMEM((FPW * N,), jnp.float32),
        pltpu.VMEM((2 * CH,), jnp.int32),
        pltpu.VMEM((2 * CH,), jnp.float32),
        pltpu.SemaphoreType.DMA,
        pltpu.SemaphoreType.DMA,
    ],
)


def kernel(x, edge_index, adj_vals, W):
    xpT = _pack(x).reshape(-1)
    packed = (edge_index[0] << SHIFT) | edge_index[1]
    hiT = _sc_call(xpT, packed, adj_vals)
    return _proj(hiT.reshape(D, N), W)


# speedup vs baseline: 1.0071x; 1.0071x over previous
import jax
import jax.numpy as jnp
from jax import lax
from jax.experimental import pallas as pl
from jax.experimental.pallas import tpu as pltpu
from jax.experimental.pallas import tpu_sc as plsc

N = 10000
E = 320000
D = 128

NC = 2
NS = 16
LANES = 16
NW = NC * NS
FPW = D // NW
PPW = FPW // 2
CH = 16000
NCHUNK = E // CH
GROUPS = CH // LANES
SHIFT = 14
MASK = (1 << SHIFT) - 1


def _pack_body(x_ref, o_ref):
    xb = x_ref[...].astype(jnp.bfloat16)
    u = lax.bitcast_convert_type(xb, jnp.uint16)
    lo = u[:, : D // 2].astype(jnp.uint32)
    hi = u[:, D // 2:].astype(jnp.uint32)
    word = lax.bitcast_convert_type(lo | (hi << 16), jnp.int32)
    o_ref[...] = word.T


def _pack(x):
    return pl.pallas_call(
        _pack_body,
        out_shape=jax.ShapeDtypeStruct((D // 2, N), jnp.int32),
    )(x)


def _proj_body(h_ref, w_ref, o_ref):
    o_ref[...] = lax.dot_general(
        h_ref[...], w_ref[...], (((0,), (0,)), ((), ())),
        preferred_element_type=jnp.float32)


def _proj(hiT, W):
    return pl.pallas_call(
        _proj_body,
        out_shape=jax.ShapeDtypeStruct((N, D), jnp.float32),
    )(hiT, W)


def _sc_body(xp_hbm, packed_hbm, vals_hbm, out_hbm,
             xcols, acc, pk_b, vals_b, semA, semB):
    w = lax.axis_index("s") * NC + lax.axis_index("c")
    pltpu.sync_copy(xp_hbm.at[pl.ds(w * PPW * N, PPW * N)], xcols)

    zeros = jnp.zeros((LANES,), jnp.float32)

    @plsc.parallel_loop(0, FPW * N // LANES, unroll=8)
    def _zero(i):
        acc[pl.ds(i * LANES, LANES)] = zeros

    poff = [jnp.full((LANES,), p * N, jnp.int32) for p in range(PPW)]
    coff = [jnp.full((LANES,), c * N, jnp.int32) for c in range(FPW)]
    sems = (semA, semB)

    def _start(ck, b, sem):
        off = ck * CH
        pltpu.async_copy(packed_hbm.at[pl.ds(off, CH)],
                         pk_b.at[pl.ds(b * CH, CH)], sem)
        pltpu.async_copy(vals_hbm.at[pl.ds(off, CH)],
                         vals_b.at[pl.ds(b * CH, CH)], sem)

    def _drain(b, sem):
        pltpu.make_async_copy(packed_hbm.at[pl.ds(0, CH)],
                              pk_b.at[pl.ds(b * CH, CH)], sem).wait()
        pltpu.make_async_copy(vals_hbm.at[pl.ds(0, CH)],
                              vals_b.at[pl.ds(b * CH, CH)], sem).wait()

    _start(0, 0, semA)
    _start(1, 1, semB)

    def pair_loop(p, carry):
        for b in range(2):
            ck = p * 2 + b
            sem = sems[b]
            _drain(b, sem)

            @plsc.parallel_loop(0, GROUPS, unroll=4)
            def _group(g):
                base = b * CH + g * LANES
                p16 = pk_b[pl.ds(base, LANES)]
                v16 = vals_b[pl.ds(base, LANES)]
                s16 = p16 & MASK
                d16 = lax.shift_right_logical(p16, SHIFT)
                for cp in range(PPW):
                    gw = plsc.load_gather(xcols, [s16 + poff[cp]])
                    f0 = plsc.bitcast(lax.shift_left(gw, 16), jnp.float32)
                    f1 = plsc.bitcast(gw & jnp.int32(-65536), jnp.float32)
                    plsc.addupdate_scatter(acc, [d16 + coff[cp]],
                                           v16 * f0)
                    plsc.addupdate_scatter(acc, [d16 + coff[cp + PPW]],
                                           v16 * f1)

            nxt = (ck + 2) - NCHUNK * ((ck + 2) // NCHUNK)
            _start(nxt, b, sem)
        return carry

    lax.fori_loop(0, NCHUNK // 2, pair_loop, 0)
    _drain(0, semA)
    _drain(1, semB)
    pltpu.sync_copy(acc.at[pl.ds(0, PPW * N)],
                    out_hbm.at[pl.ds(w * PPW * N, PPW * N)])
    pltpu.sync_copy(acc.at[pl.ds(PPW * N, PPW * N)],
                    out_hbm.at[pl.ds((D // 2 + w * PPW) * N, PPW * N)])


_sc_call = pl.kernel(
    _sc_body,
    out_type=jax.ShapeDtypeStruct((D * N,), jnp.float32),
    mesh=plsc.VectorSubcoreMesh(core_axis_name="c", subcore_axis_name="s",
                                num_cores=NC, num_subcores=NS),
    compiler_params=pltpu.CompilerParams(needs_layout_passes=False),
    scratch_types=[
        pltpu.VMEM((PPW * N,), jnp.int32),
        pltpu.VMEM((FPW * N,), jnp.float32),
        pltpu.VMEM((2 * CH,), jnp.int32),
        pltpu.VMEM((2 * CH,), jnp.float32),
        pltpu.SemaphoreType.DMA,
        pltpu.SemaphoreType.DMA,
    ],
)


def kernel(x, edge_index, adj_vals, W):
    xpT = _pack(x).reshape(-1)
    packed = (edge_index[0] << SHIFT) | edge_index[1]
    hiT = _sc_call(xpT, packed, adj_vals)
    return _proj(hiT.reshape(D, N), W)


# speedup vs baseline: 1.0116x; 1.0045x over previous
import jax
import jax.numpy as jnp
from jax import lax
from jax.experimental import pallas as pl
from jax.experimental.pallas import tpu as pltpu
from jax.experimental.pallas import tpu_sc as plsc

N = 10000
E = 320000
D = 128

NC = 2
NS = 16
LANES = 16
NW = NC * NS
FPW = D // NW
PPW = FPW // 2
CH = 16000
NCHUNK = E // CH
GROUPS = CH // LANES
SHIFT = 14
MASK = (1 << SHIFT) - 1


def _pack_body(x_ref, o_ref):
    xb = x_ref[...].astype(jnp.bfloat16)
    u = lax.bitcast_convert_type(xb, jnp.uint16)
    lo = u[:, : D // 2].astype(jnp.uint32)
    hi = u[:, D // 2:].astype(jnp.uint32)
    word = lax.bitcast_convert_type(lo | (hi << 16), jnp.int32)
    o_ref[...] = word.T


def _pack(x):
    return pl.pallas_call(
        _pack_body,
        out_shape=jax.ShapeDtypeStruct((D // 2, N), jnp.int32),
    )(x)


def _proj_body(h_ref, w_ref, o_ref):
    o_ref[...] = lax.dot_general(
        h_ref[...], w_ref[...], (((0,), (0,)), ((), ())),
        preferred_element_type=jnp.float32)


def _proj(hiT, W):
    return pl.pallas_call(
        _proj_body,
        out_shape=jax.ShapeDtypeStruct((N, D), jnp.float32),
    )(hiT, W)


def _sc_body(xp_hbm, packed_hbm, vals_hbm, out_hbm,
             xcols, acc, pk_b, vals_b, semA, semB):
    w = lax.axis_index("s") * NC + lax.axis_index("c")

    poff = [jnp.full((LANES,), p * N, jnp.int32) for p in range(PPW)]
    coff = [jnp.full((LANES,), c * N, jnp.int32) for c in range(FPW)]
    sems = (semA, semB)

    def _start(ck, b, sem):
        off = ck * CH
        pltpu.async_copy(packed_hbm.at[pl.ds(off, CH)],
                         pk_b.at[pl.ds(b * CH, CH)], sem)
        pltpu.async_copy(vals_hbm.at[pl.ds(off, CH)],
                         vals_b.at[pl.ds(b * CH, CH)], sem)

    def _drain(b, sem):
        pltpu.make_async_copy(packed_hbm.at[pl.ds(0, CH)],
                              pk_b.at[pl.ds(b * CH, CH)], sem).wait()
        pltpu.make_async_copy(vals_hbm.at[pl.ds(0, CH)],
                              vals_b.at[pl.ds(b * CH, CH)], sem).wait()

    _start(0, 0, semA)
    _start(1, 1, semB)
    pltpu.sync_copy(xp_hbm.at[pl.ds(w * PPW * N, PPW * N)], xcols)

    zeros = jnp.zeros((LANES,), jnp.float32)

    @plsc.parallel_loop(0, FPW * N // LANES, unroll=8)
    def _zero(i):
        acc[pl.ds(i * LANES, LANES)] = zeros

    def pair_loop(p, carry):
        for b in range(2):
            ck = p * 2 + b
            sem = sems[b]
            _drain(b, sem)

            @plsc.parallel_loop(0, GROUPS, unroll=4)
            def _group(g):
                base = b * CH + g * LANES
                p16 = pk_b[pl.ds(base, LANES)]
                v16 = vals_b[pl.ds(base, LANES)]
                s16 = p16 & MASK
                d16 = lax.shift_right_logical(p16, SHIFT)
                for cp in range(PPW):
                    gw = plsc.load_gather(xcols, [s16 + poff[cp]])
                    f0 = plsc.bitcast(lax.shift_left(gw, 16), jnp.float32)
                    f1 = plsc.bitcast(gw & jnp.int32(-65536), jnp.float32)
                    plsc.addupdate_scatter(acc, [d16 + coff[cp]],
                                           v16 * f0)
                    plsc.addupdate_scatter(acc, [d16 + coff[cp + PPW]],
                                           v16 * f1)

            nxt = (ck + 2) - NCHUNK * ((ck + 2) // NCHUNK)
            _start(nxt, b, sem)
        return carry

    lax.fori_loop(0, NCHUNK // 2, pair_loop, 0)
    _drain(0, semA)
    _drain(1, semB)
    pltpu.sync_copy(acc.at[pl.ds(0, PPW * N)],
                    out_hbm.at[pl.ds(w * PPW * N, PPW * N)])
    pltpu.sync_copy(acc.at[pl.ds(PPW * N, PPW * N)],
                    out_hbm.at[pl.ds((D // 2 + w * PPW) * N, PPW * N)])


_sc_call = pl.kernel(
    _sc_body,
    out_type=jax.ShapeDtypeStruct((D * N,), jnp.float32),
    mesh=plsc.VectorSubcoreMesh(core_axis_name="c", subcore_axis_name="s",
                                num_cores=NC, num_subcores=NS),
    compiler_params=pltpu.CompilerParams(needs_layout_passes=False),
    scratch_types=[
        pltpu.VMEM((PPW * N,), jnp.int32),
        pltpu.VMEM((FPW * N,), jnp.float32),
        pltpu.VMEM((2 * CH,), jnp.int32),
        pltpu.VMEM((2 * CH,), jnp.float32),
        pltpu.SemaphoreType.DMA,
        pltpu.SemaphoreType.DMA,
    ],
)


def kernel(x, edge_index, adj_vals, W):
    xpT = _pack(x).reshape(-1)
    packed = (edge_index[0] << SHIFT) | edge_index[1]
    hiT = _sc_call(xpT, packed, adj_vals)
    return _proj(hiT.reshape(D, N), W)
